# Initial kernel scaffold; baseline (speedup 1.0000x reference)
import jax
import jax.numpy as jnp
from jax.experimental import pallas as pl


def kernel(synonym_outputs, predictions, labels, original_sentence, perturbed_sentence, embedding_table):
    raise NotImplementedError("write your pallas kernel here")


# speedup vs baseline: 4.7283x; 4.7283x over previous
import functools

import jax
import jax.numpy as jnp
from jax import lax
from jax.experimental import pallas as pl
from jax.experimental.pallas import tpu as pltpu
from jax.experimental.pallas import tpu_sc as plsc

_B, _S, _V, _D = 1024, 200, 100000, 128
_N = _B * _S
_NC, _NS, _L = 2, 16, 16
_NW = _NC * _NS
_PER_W = _N // _NW
_T = 128
_NCHUNK = _PER_W // _T
_KAPPA = 5.0


def _rsqrt16(s):
    i = plsc.bitcast(s, jnp.int32)
    i = jnp.int32(0x5F3759DF) - (i >> 1)
    y = plsc.bitcast(i, jnp.float32)
    for _ in range(3):
        y = y * (jnp.float32(1.5) - jnp.float32(0.5) * s * y * y)
    return y


def _sc_cosine_partials(table, orig_idx, pert_idx):
    mesh = plsc.VectorSubcoreMesh(core_axis_name="c", subcore_axis_name="s")

    @functools.partial(
        pl.kernel,
        mesh=mesh,
        out_type=jax.ShapeDtypeStruct((_NW * _L,), jnp.float32),
        compiler_params=pltpu.CompilerParams(needs_layout_passes=False),
        scratch_types=[
            pltpu.VMEM((_T,), jnp.int32),
            pltpu.VMEM((_T,), jnp.int32),
            pltpu.VMEM((_T, _D), jnp.float32),
            pltpu.VMEM((_T, _D), jnp.float32),
            pltpu.VMEM((_T,), jnp.float32),
            pltpu.VMEM((_T,), jnp.float32),
            pltpu.VMEM((_T,), jnp.float32),
            pltpu.VMEM((_L,), jnp.float32),
            pltpu.SemaphoreType.DMA,
            pltpu.SemaphoreType.DMA,
        ],
    )
    def sc_kernel(table_hbm, oidx_hbm, pidx_hbm, out_hbm,
                  idx_o, idx_p, o_rows, p_rows, dot_b, no2_b, np2_b,
                  acc_b, sem0, sem1):
        wid = lax.axis_index("s") * _NC + lax.axis_index("c")
        base = wid * _PER_W
        lanes = lax.iota(jnp.int32, _L)
        last = lanes == (_L - 1)

        def chunk(c, acc):
            off = base + c * _T
            pltpu.sync_copy(oidx_hbm.at[pl.ds(off, _T)], idx_o)
            pltpu.sync_copy(pidx_hbm.at[pl.ds(off, _T)], idx_p)
            cp0 = pltpu.async_copy(table_hbm.at[idx_o], o_rows, sem0)
            cp1 = pltpu.async_copy(table_hbm.at[idx_p], p_rows, sem1)
            cp0.wait()
            cp1.wait()

            def tok(t, carry):
                accd = acco = accp = None
                for g in range(_D // _L):
                    o = o_rows[t, pl.ds(g * _L, _L)]
                    p = p_rows[t, pl.ds(g * _L, _L)]
                    if g == 0:
                        accd, acco, accp = o * p, o * o, p * p
                    else:
                        accd = accd + o * p
                        acco = acco + o * o
                        accp = accp + p * p
                tt = jnp.full((_L,), t, dtype=jnp.int32)
                plsc.store_scatter(dot_b, [tt], plsc.cumsum(accd), mask=last)
                plsc.store_scatter(no2_b, [tt], plsc.cumsum(acco), mask=last)
                plsc.store_scatter(np2_b, [tt], plsc.cumsum(accp), mask=last)
                return carry

            lax.fori_loop(0, _T, tok, 0)

            for j in range(_T // _L):
                dv = dot_b[pl.ds(j * _L, _L)]
                ov = no2_b[pl.ds(j * _L, _L)]
                pv = np2_b[pl.ds(j * _L, _L)]
                acc = acc + dv * _rsqrt16(ov * pv)
            return acc

        acc = lax.fori_loop(0, _NCHUNK, chunk, jnp.zeros((_L,), jnp.float32))
        acc_b[...] = acc
        pltpu.sync_copy(acc_b, out_hbm.at[pl.ds(wid * _L, _L)])

    return sc_kernel(table, orig_idx, pert_idx)


def _tc_combine(pred, lab, syn, sc_part):
    def body(pred_ref, lab_ref, syn_ref, sc_ref, out_ref):
        p = pred_ref[...]
        l = lab_ref[...]
        take1 = l[:, 1:2] > l[:, 0:1]
        correct = jnp.where(take1, p[:, 1:2], p[:, 0:1])
        incorrect = jnp.where(take1, p[:, 0:1], p[:, 1:2])
        adv = jnp.sum(jnp.maximum(correct - incorrect + _KAPPA, 0.0)) / _B

        cos = jnp.sum(sc_ref[...]) / _N

        syn_mean = jnp.sum(syn_ref[...]) / _B
        synl = (syn_mean - 50.0) * (syn_mean - 50.0) + 1.0

        loss = adv - cos + synl
        col = lax.broadcasted_iota(jnp.int32, (8, 128), 1)
        out_ref[...] = jnp.where(
            col == 0, loss,
            jnp.where(col == 1, adv,
                      jnp.where(col == 2, cos,
                                jnp.where(col == 3, synl, 0.0))))

    return pl.pallas_call(
        body,
        out_shape=jax.ShapeDtypeStruct((8, 128), jnp.float32),
    )(pred, lab, syn, sc_part)


def kernel(synonym_outputs, predictions, labels, original_sentence,
           perturbed_sentence, embedding_table):
    oidx = original_sentence.reshape(-1)
    pidx = perturbed_sentence.reshape(-1)
    sc_part = _sc_cosine_partials(embedding_table, oidx, pidx)
    out = _tc_combine(predictions, labels, synonym_outputs,
                      sc_part.reshape(4, 128))
    return (out[0, 0], out[0, 1], out[0, 2], out[0, 3])
